# baseline (device time: 379290 ns/iter reference)
import jax
import jax.numpy as jnp
from jax import lax
from jax.experimental import pallas as pl
from jax.experimental.pallas import tpu as pltpu

N_DEV = 4
HQ = 32
HL = 8
DH = 128
SQ = 2048
SKV_SH = 2048
SKV = 8192
BQ = 256
HG = 2
SCALE = 0.08838834764831843
BF16 = jnp.bfloat16
MESH = pl.DeviceIdType.MESH


_CVT_CHUNK = 512


def _convert_body(k_ref, v_ref, kt_ref, vt_ref):
    kt_ref[...] = jnp.transpose(k_ref[0].astype(BF16), (1, 0, 2))
    vt_ref[...] = jnp.transpose(v_ref[0].astype(BF16), (1, 0, 2))


def _convert(K_ext, V_ext):
    return pl.pallas_call(
        _convert_body,
        grid=(N_DEV, SKV_SH // _CVT_CHUNK),
        in_specs=[
            pl.BlockSpec((1, _CVT_CHUNK, HL, DH), lambda g, c: (0, c, g, 0)),
            pl.BlockSpec((1, _CVT_CHUNK, HL, DH), lambda g, c: (0, c, g, 0)),
        ],
        out_specs=[
            pl.BlockSpec((HL, _CVT_CHUNK, DH), lambda g, c: (g, c, 0)),
            pl.BlockSpec((HL, _CVT_CHUNK, DH), lambda g, c: (g, c, 0)),
        ],
        out_shape=[
            jax.ShapeDtypeStruct((HQ, SKV_SH, DH), BF16),
            jax.ShapeDtypeStruct((HQ, SKV_SH, DH), BF16),
        ],
    )(K_ext, V_ext)


def _attend(qblk, qi0, segs):
    parts = [
        lax.dot_general(qblk, k, (((1,), (1,)), ((), ())),
                        preferred_element_type=jnp.float32)
        for k, _, _ in segs
    ]
    s = jnp.concatenate(parts, axis=1) * SCALE if len(parts) > 1 else parts[0] * SCALE
    ki = jnp.concatenate(
        [ki0 + lax.broadcasted_iota(jnp.int32, (1, k.shape[0]), 1)
         for k, _, ki0 in segs], axis=1)
    qi = qi0 + lax.broadcasted_iota(jnp.int32, s.shape, 0)
    mask = (jnp.abs(qi - ki) <= 128) | (ki < 32) | (qi < 32)
    s = jnp.where(mask, s, -1e9)
    e = jnp.exp(s - jnp.max(s, axis=1, keepdims=True))
    w = (e / jnp.sum(e, axis=1, keepdims=True)).astype(BF16)
    acc = None
    off = 0
    for k, v, _ in segs:
        p = jnp.dot(w[:, off:off + k.shape[0]], v,
                    preferred_element_type=jnp.float32)
        acc = p if acc is None else acc + p
        off += k.shape[0]
    return acc


def _attn_body(x_ref, wq_ref, wo_ref, kt_ref, vt_ref,
               out_ref,
               q_buf, ctx_buf, k0_buf, v0_buf, k1s_buf, v1s_buf,
               qg_buf, sout, lout, sin, lin, kst, vst,
               qg_send, qg_recv, k0_send, k0_recv, v0_send, v0_recv,
               k1_send, k1_recv, v1_send, v1_recv,
               stS_send, stS_recv, stL_send, stL_recv,
               cp_sem, ld_sem):
    h = pl.program_id(0)
    me = lax.axis_index("i")

    @pl.when(h == 0)
    def _comm():
        barrier = pltpu.get_barrier_semaphore()
        for d in range(1, N_DEV):
            pl.semaphore_signal(barrier, inc=1,
                                device_id=((me + d) % N_DEV,),
                                device_id_type=MESH)
        pl.semaphore_wait(barrier, N_DEV - 1)

        waiters = []

        xg = x_ref[0, 0:32, :].astype(BF16)
        qg_buf[me] = jnp.dot(xg, wq_ref[...].astype(BF16),
                             preferred_element_type=jnp.float32).astype(BF16)
        for d in range(1, N_DEV):
            dst = (me + d) % N_DEV
            r = pltpu.make_async_remote_copy(
                src_ref=qg_buf.at[me], dst_ref=qg_buf.at[me],
                send_sem=qg_send.at[d - 1], recv_sem=qg_recv.at[me],
                device_id=(dst,), device_id_type=MESH)
            r.start()
            waiters.append(r)

        @pl.when(me == 0)
        def _send_chunk0():
            for d in range(1, N_DEV):
                for g in range(HL // HG):
                    for src_t, dbuf, ssem, rsem in (
                        (kt_ref, k0_buf, k0_send, k0_recv),
                        (vt_ref, v0_buf, v0_send, v0_recv),
                    ):
                        r = pltpu.make_async_remote_copy(
                            src_ref=src_t.at[pl.ds(HL * d + g * HG, HG)],
                            dst_ref=dbuf.at[pl.ds(g * HG, HG)],
                            send_sem=ssem.at[(d - 1) * (HL // HG) + g],
                            recv_sem=rsem.at[g],
                            device_id=(d,), device_id_type=MESH)
                        r.start()
            pltpu.make_async_copy(
                kt_ref.at[pl.ds(0, HL)], k0_buf, cp_sem.at[0]).start()
            pltpu.make_async_copy(
                vt_ref.at[pl.ds(0, HL)], v0_buf, cp_sem.at[1]).start()

        @pl.when(me == 1)
        def _send_sliver():
            for d in range(1, N_DEV):
                dst = (1 + d) % N_DEV
                for src_t, dbuf, ssem, rsem in (
                    (kt_ref, k1s_buf, k1_send, k1_recv),
                    (vt_ref, v1s_buf, v1_send, v1_recv),
                ):
                    r = pltpu.make_async_remote_copy(
                        src_ref=src_t.at[pl.ds(HL * dst, HL), pl.ds(0, 128)],
                        dst_ref=dbuf,
                        send_sem=ssem.at[d - 1], recv_sem=rsem,
                        device_id=(dst,), device_id_type=MESH)
                    r.start()
            pltpu.make_async_copy(
                kt_ref.at[pl.ds(HL, HL), pl.ds(0, 128)], k1s_buf,
                cp_sem.at[0]).start()
            pltpu.make_async_copy(
                vt_ref.at[pl.ds(HL, HL), pl.ds(0, 128)], v1s_buf,
                cp_sem.at[1]).start()

        for rb in range(4):
            rows = pl.ds(rb * 512, 512)
            q = jnp.dot(x_ref[0, rows, :].astype(BF16),
                        wq_ref[...].astype(BF16),
                        preferred_element_type=jnp.float32)
            q_buf[rows, :] = q.astype(BF16)

        for d in range(1, N_DEV):
            src = (me + d) % N_DEV
            pltpu.make_async_remote_copy(
                src_ref=qg_buf.at[me], dst_ref=qg_buf.at[me],
                send_sem=qg_send.at[d - 1], recv_sem=qg_recv.at[src],
                device_id=(src,), device_id_type=MESH).wait_recv()

        for r in range(N_DEV):
            ck = pltpu.make_async_copy(
                kt_ref.at[pl.ds(HL * r, HL)], kst, ld_sem.at[0])
            cv = pltpu.make_async_copy(
                vt_ref.at[pl.ds(HL * r, HL)], vst, ld_sem.at[1])
            ck.start()
            cv.start()
            ck.wait()
            cv.wait()
            qr = jnp.transpose(qg_buf[r].reshape(32, HL, DH), (1, 0, 2))
            s = lax.dot_general(qr, kst[...],
                                (((2,), (2,)), ((0,), (0,))),
                                preferred_element_type=jnp.float32) * SCALE
            e = jnp.exp(s)
            lsum = jnp.sum(e, axis=2)
            S = lax.dot_general(e.astype(BF16), vst[...],
                                (((2,), (1,)), ((0,), (0,))),
                                preferred_element_type=jnp.float32)
            sout[r] = S
            lout[r] = lsum

            @pl.when(r == me)
            def _keep_own():
                sin[me] = S
                lin[me] = lsum

            @pl.when(r != me)
            def _send_stats():
                rs = pltpu.make_async_remote_copy(
                    src_ref=sout.at[r], dst_ref=sin.at[me],
                    send_sem=stS_send.at[r], recv_sem=stS_recv.at[me],
                    device_id=(r,), device_id_type=MESH)
                rl = pltpu.make_async_remote_copy(
                    src_ref=lout.at[r], dst_ref=lin.at[me],
                    send_sem=stL_send.at[r],
                    recv_sem=stL_recv.at[me],
                    device_id=(r,), device_id_type=MESH)
                rs.start()
                rl.start()

        for d in range(1, N_DEV):
            src = (me + d) % N_DEV
            pltpu.make_async_remote_copy(
                src_ref=sout.at[0], dst_ref=sin.at[src],
                send_sem=stS_send.at[0], recv_sem=stS_recv.at[src],
                device_id=(src,), device_id_type=MESH).wait_recv()
            pltpu.make_async_remote_copy(
                src_ref=lout.at[0], dst_ref=lin.at[src],
                send_sem=stL_send.at[0], recv_sem=stL_recv.at[src],
                device_id=(src,), device_id_type=MESH).wait_recv()

        @pl.when(me != 1)
        def _wait_sliver():
            pltpu.make_async_remote_copy(
                src_ref=kt_ref.at[pl.ds(0, HL), pl.ds(0, 128)],
                dst_ref=k1s_buf,
                send_sem=k1_send.at[0], recv_sem=k1_recv,
                device_id=(1,), device_id_type=MESH).wait_recv()
            pltpu.make_async_remote_copy(
                src_ref=vt_ref.at[pl.ds(0, HL), pl.ds(0, 128)],
                dst_ref=v1s_buf,
                send_sem=v1_send.at[0], recv_sem=v1_recv,
                device_id=(1,), device_id_type=MESH).wait_recv()

        @pl.when(me == 0)
        def _wait_own_chunk0():
            pltpu.make_async_copy(
                kt_ref.at[pl.ds(0, HL)], k0_buf, cp_sem.at[0]).wait()
            pltpu.make_async_copy(
                vt_ref.at[pl.ds(0, HL)], v0_buf, cp_sem.at[1]).wait()

        @pl.when(me == 1)
        def _wait_own_sliver():
            pltpu.make_async_copy(
                kt_ref.at[pl.ds(HL, HL), pl.ds(0, 128)], k1s_buf,
                cp_sem.at[0]).wait()
            pltpu.make_async_copy(
                vt_ref.at[pl.ds(HL, HL), pl.ds(0, 128)], v1s_buf,
                cp_sem.at[1]).wait()

        for r in waiters:
            r.wait_send()
        for d in range(1, N_DEV):
            dst = (me + d) % N_DEV
            pltpu.make_async_remote_copy(
                src_ref=sout.at[dst], dst_ref=sin.at[me],
                send_sem=stS_send.at[dst], recv_sem=stS_recv.at[me],
                device_id=(dst,), device_id_type=MESH).wait_send()
            pltpu.make_async_remote_copy(
                src_ref=lout.at[dst], dst_ref=lin.at[me],
                send_sem=stL_send.at[dst], recv_sem=stL_recv.at[me],
                device_id=(dst,), device_id_type=MESH).wait_send()

        Ssum = sin[0] + sin[1] + sin[2] + sin[3]
        Lsum = lin[0] + lin[1] + lin[2] + lin[3]
        ctx_g = (Ssum / Lsum[:, :, None]).astype(BF16)
        for hh in range(HL):
            ctx_buf[pl.ds(0, 32), pl.ds(DH * hh, DH)] = ctx_g[hh]

    @pl.when((me != 0) & (h % HG == 0))
    def _wait_k0_group():
        pltpu.make_async_remote_copy(
            src_ref=kt_ref.at[pl.ds(0, HG)],
            dst_ref=k0_buf.at[pl.ds(h, HG)],
            send_sem=k0_send.at[0], recv_sem=k0_recv.at[h // HG],
            device_id=(0,), device_id_type=MESH).wait_recv()
        pltpu.make_async_remote_copy(
            src_ref=vt_ref.at[pl.ds(0, HG)],
            dst_ref=v0_buf.at[pl.ds(h, HG)],
            send_sem=v0_send.at[0], recv_sem=v0_recv.at[h // HG],
            device_id=(0,), device_id_type=MESH).wait_recv()

    k0h = k0_buf[h]
    v0h = v0_buf[h]
    k1h = k1s_buf[h]
    v1h = v1s_buf[h]
    qh = q_buf[:, pl.ds(DH * h, DH)]
    col = pl.ds(DH * h, DH)

    ctx_buf[pl.ds(32, 224), col] = _attend(
        qh[32:256], 32, [(k0h[:512], v0h[:512], 0)]).astype(BF16)
    ctx_buf[pl.ds(256, 256), col] = _attend(
        qh[256:512], 256, [(k0h[:768], v0h[:768], 0)]).astype(BF16)
    for qb in range(2, 7):
        lo = (qb - 1) * BQ
        ctx_buf[pl.ds(qb * BQ, BQ), col] = _attend(
            qh[qb * BQ:(qb + 1) * BQ], qb * BQ,
            [(k0h[:BQ], v0h[:BQ], 0),
             (k0h[lo:lo + 3 * BQ], v0h[lo:lo + 3 * BQ], lo)],
        ).astype(BF16)
    ctx_buf[pl.ds(7 * BQ, BQ), col] = _attend(
        qh[7 * BQ:8 * BQ], 7 * BQ,
        [(k0h[:BQ], v0h[:BQ], 0),
         (k0h[6 * BQ:8 * BQ], v0h[6 * BQ:8 * BQ], 6 * BQ),
         (k1h, v1h, 2048)],
    ).astype(BF16)

    @pl.when((h == HL - 1) & (me == 0))
    def _drain_chunk0_sends():
        for d in range(1, N_DEV):
            for g in range(HL // HG):
                pltpu.make_async_remote_copy(
                    src_ref=kt_ref.at[pl.ds(HL * d + g * HG, HG)],
                    dst_ref=k0_buf.at[pl.ds(g * HG, HG)],
                    send_sem=k0_send.at[(d - 1) * (HL // HG) + g],
                    recv_sem=k0_recv.at[g],
                    device_id=(d,), device_id_type=MESH).wait_send()
                pltpu.make_async_remote_copy(
                    src_ref=vt_ref.at[pl.ds(HL * d + g * HG, HG)],
                    dst_ref=v0_buf.at[pl.ds(g * HG, HG)],
                    send_sem=v0_send.at[(d - 1) * (HL // HG) + g],
                    recv_sem=v0_recv.at[g],
                    device_id=(d,), device_id_type=MESH).wait_send()

    @pl.when((h == HL - 1) & (me == 1))
    def _drain_sliver_sends():
        for d in range(1, N_DEV):
            dst = (1 + d) % N_DEV
            pltpu.make_async_remote_copy(
                src_ref=kt_ref.at[pl.ds(HL * dst, HL), pl.ds(0, 128)],
                dst_ref=k1s_buf,
                send_sem=k1_send.at[d - 1], recv_sem=k1_recv,
                device_id=(dst,), device_id_type=MESH).wait_send()
            pltpu.make_async_remote_copy(
                src_ref=vt_ref.at[pl.ds(HL * dst, HL), pl.ds(0, 128)],
                dst_ref=v1s_buf,
                send_sem=v1_send.at[d - 1], recv_sem=v1_recv,
                device_id=(dst,), device_id_type=MESH).wait_send()

    @pl.when(h == HL - 1)
    def _project_out():
        out = jnp.dot(ctx_buf[...], wo_ref[...].astype(BF16),
                      preferred_element_type=jnp.float32)
        out_ref[...] = out.astype(BF16)


def _attn(x, Wq, Wo, kt, vt):
    return pl.pallas_call(
        _attn_body,
        grid=(HL,),
        in_specs=[
            pl.BlockSpec((1, SQ, 1024), lambda h: (0, 0, 0)),
            pl.BlockSpec((1024, 1024), lambda h: (0, 0)),
            pl.BlockSpec((1024, 1024), lambda h: (0, 0)),
            pl.BlockSpec(memory_space=pl.ANY),
            pl.BlockSpec(memory_space=pl.ANY),
        ],
        out_specs=pl.BlockSpec((SQ, 1024), lambda h: (0, 0)),
        out_shape=jax.ShapeDtypeStruct((SQ, 1024), BF16),
        scratch_shapes=[
            pltpu.VMEM((SQ, HL * DH), BF16),
            pltpu.VMEM((SQ, HL * DH), BF16),
            pltpu.VMEM((HL, SKV_SH, DH), BF16),
            pltpu.VMEM((HL, SKV_SH, DH), BF16),
            pltpu.VMEM((HL, 128, DH), BF16),
            pltpu.VMEM((HL, 128, DH), BF16),
            pltpu.VMEM((N_DEV, 32, HL * DH), BF16),
            pltpu.VMEM((N_DEV, HL, 32, DH), jnp.float32),
            pltpu.VMEM((N_DEV, HL, 32), jnp.float32),
            pltpu.VMEM((N_DEV, HL, 32, DH), jnp.float32),
            pltpu.VMEM((N_DEV, HL, 32), jnp.float32),
            pltpu.VMEM((HL, SKV_SH, DH), BF16),
            pltpu.VMEM((HL, SKV_SH, DH), BF16),
            pltpu.SemaphoreType.DMA((N_DEV - 1,)),
            pltpu.SemaphoreType.DMA((N_DEV,)),
            pltpu.SemaphoreType.DMA(((N_DEV - 1) * (HL // HG),)),
            pltpu.SemaphoreType.DMA((HL // HG,)),
            pltpu.SemaphoreType.DMA(((N_DEV - 1) * (HL // HG),)),
            pltpu.SemaphoreType.DMA((HL // HG,)),
            pltpu.SemaphoreType.DMA((N_DEV - 1,)),
            pltpu.SemaphoreType.DMA,
            pltpu.SemaphoreType.DMA((N_DEV - 1,)),
            pltpu.SemaphoreType.DMA,
            pltpu.SemaphoreType.DMA((N_DEV,)),
            pltpu.SemaphoreType.DMA((N_DEV,)),
            pltpu.SemaphoreType.DMA((N_DEV,)),
            pltpu.SemaphoreType.DMA((N_DEV,)),
            pltpu.SemaphoreType.DMA((2,)),
            pltpu.SemaphoreType.DMA((2,)),
        ],
        compiler_params=pltpu.CompilerParams(
            collective_id=0, vmem_limit_bytes=56 * 1024 * 1024),
    )(x, Wq, Wo, kt, vt)


_CH = SQ // N_DEV


def _ar_body(p_ref, out_ref, rbuf, sbuf, send_s, recv_s):
    me = lax.axis_index("i")
    left = (me - 1) % N_DEV
    right = (me + 1) % N_DEV

    barrier = pltpu.get_barrier_semaphore()
    for nbr in (left, right):
        pl.semaphore_signal(barrier, inc=1, device_id=(nbr,),
                            device_id_type=MESH)
    pl.semaphore_wait(barrier, 2)

    def chunk(ref, c):
        return ref[pl.ds(_CH * c, _CH), :]

    def hop(src_ref, t):
        r = pltpu.make_async_remote_copy(
            src_ref=src_ref,
            dst_ref=rbuf.at[t],
            send_sem=send_s.at[t],
            recv_sem=recv_s.at[t],
            device_id=(right,),
            device_id_type=MESH,
        )
        r.start()
        r.wait()

    hop(p_ref.at[pl.ds(_CH * me, _CH)], 0)
    s = rbuf[0].astype(jnp.float32) + chunk(p_ref, (me - 1) % N_DEV).astype(jnp.float32)
    sbuf[0] = s.astype(BF16)
    hop(sbuf.at[0], 1)
    s = rbuf[1].astype(jnp.float32) + chunk(p_ref, (me - 2) % N_DEV).astype(jnp.float32)
    sbuf[1] = s.astype(BF16)
    hop(sbuf.at[1], 2)
    f = rbuf[2].astype(jnp.float32) + chunk(p_ref, (me + 1) % N_DEV).astype(jnp.float32)
    sbuf[2] = f.astype(BF16)
    out_ref[0, pl.ds(_CH * ((me + 1) % N_DEV), _CH), :] = f

    hop(sbuf.at[2], 3)
    out_ref[0, pl.ds(_CH * me, _CH), :] = rbuf[3].astype(jnp.float32)
    hop(rbuf.at[3], 4)
    out_ref[0, pl.ds(_CH * ((me - 1) % N_DEV), _CH), :] = rbuf[4].astype(jnp.float32)
    hop(rbuf.at[4], 5)
    out_ref[0, pl.ds(_CH * ((me - 2) % N_DEV), _CH), :] = rbuf[5].astype(jnp.float32)


def _allreduce(partial):
    return pl.pallas_call(
        _ar_body,
        in_specs=[pl.BlockSpec(memory_space=pltpu.VMEM)],
        out_specs=pl.BlockSpec(memory_space=pltpu.VMEM),
        out_shape=jax.ShapeDtypeStruct((1, SQ, 1024), jnp.float32),
        scratch_shapes=[
            pltpu.VMEM((6, _CH, 1024), BF16),
            pltpu.VMEM((3, _CH, 1024), BF16),
            pltpu.SemaphoreType.DMA((6,)),
            pltpu.SemaphoreType.DMA((6,)),
        ],
        compiler_params=pltpu.CompilerParams(
            collective_id=1, vmem_limit_bytes=48 * 1024 * 1024),
    )(partial)


def kernel(x, Wq, K_ext, V_ext, Wo):
    kt, vt = _convert(K_ext, V_ext)
    partial = _attn(x, Wq, Wo, kt, vt)
    return _allreduce(partial)


# device time: 376046 ns/iter; 1.0086x vs baseline; 1.0086x over previous
import jax
import jax.numpy as jnp
from jax import lax
from jax.experimental import pallas as pl
from jax.experimental.pallas import tpu as pltpu

N_DEV = 4
HQ = 32
HL = 8
DH = 128
SQ = 2048
SKV_SH = 2048
SKV = 8192
BQ = 256
HG = 2
SCALE = 0.08838834764831843
BF16 = jnp.bfloat16
MESH = pl.DeviceIdType.MESH


_CVT_CHUNK = 512


def _convert_body(k_ref, v_ref, kt_ref, vt_ref):
    kt_ref[...] = jnp.transpose(k_ref[0].astype(BF16), (1, 0, 2))
    vt_ref[...] = jnp.transpose(v_ref[0].astype(BF16), (1, 0, 2))


def _convert(K_ext, V_ext):
    return pl.pallas_call(
        _convert_body,
        grid=(N_DEV, SKV_SH // _CVT_CHUNK),
        in_specs=[
            pl.BlockSpec((1, _CVT_CHUNK, HL, DH), lambda g, c: (0, c, g, 0)),
            pl.BlockSpec((1, _CVT_CHUNK, HL, DH), lambda g, c: (0, c, g, 0)),
        ],
        out_specs=[
            pl.BlockSpec((HL, _CVT_CHUNK, DH), lambda g, c: (g, c, 0)),
            pl.BlockSpec((HL, _CVT_CHUNK, DH), lambda g, c: (g, c, 0)),
        ],
        out_shape=[
            jax.ShapeDtypeStruct((HQ, SKV_SH, DH), BF16),
            jax.ShapeDtypeStruct((HQ, SKV_SH, DH), BF16),
        ],
    )(K_ext, V_ext)


def _attend(qblk, qi0, segs):
    parts = [
        lax.dot_general(qblk, k, (((1,), (1,)), ((), ())),
                        preferred_element_type=jnp.float32)
        for k, _, _ in segs
    ]
    s = jnp.concatenate(parts, axis=1) * SCALE if len(parts) > 1 else parts[0] * SCALE
    ki = jnp.concatenate(
        [ki0 + lax.broadcasted_iota(jnp.int32, (1, k.shape[0]), 1)
         for k, _, ki0 in segs], axis=1)
    qi = qi0 + lax.broadcasted_iota(jnp.int32, s.shape, 0)
    mask = (jnp.abs(qi - ki) <= 128) | (ki < 32) | (qi < 32)
    s = jnp.where(mask, s, -1e9)
    e = jnp.exp(s - jnp.max(s, axis=1, keepdims=True))
    w = (e / jnp.sum(e, axis=1, keepdims=True)).astype(BF16)
    acc = None
    off = 0
    for k, v, _ in segs:
        p = jnp.dot(w[:, off:off + k.shape[0]], v,
                    preferred_element_type=jnp.float32)
        acc = p if acc is None else acc + p
        off += k.shape[0]
    return acc


def _attn_body(x_ref, wq_ref, wo_ref, kt_ref, vt_ref,
               out_ref,
               q_buf, ctx_buf, k0_buf, v0_buf, k1s_buf, v1s_buf,
               qg_buf, sout, lout, sin, lin, kst, vst,
               qg_send, qg_recv, k0_send, k0_recv, v0_send, v0_recv,
               k1_send, k1_recv, v1_send, v1_recv,
               stS_send, stS_recv, stL_send, stL_recv,
               cp_sem, ld_sem):
    h = pl.program_id(0)
    me = lax.axis_index("i")

    @pl.when(h == 0)
    def _comm():
        barrier = pltpu.get_barrier_semaphore()
        for d in range(1, N_DEV):
            pl.semaphore_signal(barrier, inc=1,
                                device_id=((me + d) % N_DEV,),
                                device_id_type=MESH)
        pl.semaphore_wait(barrier, N_DEV - 1)

        waiters = []

        xg = x_ref[0, 0:32, :].astype(BF16)
        qg_buf[me] = jnp.dot(xg, wq_ref[...].astype(BF16),
                             preferred_element_type=jnp.float32).astype(BF16)
        for d in range(1, N_DEV):
            dst = (me + d) % N_DEV
            r = pltpu.make_async_remote_copy(
                src_ref=qg_buf.at[me], dst_ref=qg_buf.at[me],
                send_sem=qg_send.at[d - 1], recv_sem=qg_recv.at[me],
                device_id=(dst,), device_id_type=MESH)
            r.start()
            waiters.append(r)

        @pl.when(me == 0)
        def _send_chunk0():
            for g in range(HL // HG):
                for d in range(1, N_DEV):
                    for src_t, dbuf, ssem, rsem in (
                        (kt_ref, k0_buf, k0_send, k0_recv),
                        (vt_ref, v0_buf, v0_send, v0_recv),
                    ):
                        r = pltpu.make_async_remote_copy(
                            src_ref=src_t.at[pl.ds(HL * d + g * HG, HG)],
                            dst_ref=dbuf.at[pl.ds(g * HG, HG)],
                            send_sem=ssem.at[(d - 1) * (HL // HG) + g],
                            recv_sem=rsem.at[g],
                            device_id=(d,), device_id_type=MESH)
                        r.start()
            pltpu.make_async_copy(
                kt_ref.at[pl.ds(0, HL)], k0_buf, cp_sem.at[0]).start()
            pltpu.make_async_copy(
                vt_ref.at[pl.ds(0, HL)], v0_buf, cp_sem.at[1]).start()

        @pl.when(me == 1)
        def _send_sliver():
            for d in range(1, N_DEV):
                dst = (1 + d) % N_DEV
                for src_t, dbuf, ssem, rsem in (
                    (kt_ref, k1s_buf, k1_send, k1_recv),
                    (vt_ref, v1s_buf, v1_send, v1_recv),
                ):
                    r = pltpu.make_async_remote_copy(
                        src_ref=src_t.at[pl.ds(HL * dst, HL), pl.ds(0, 128)],
                        dst_ref=dbuf,
                        send_sem=ssem.at[d - 1], recv_sem=rsem,
                        device_id=(dst,), device_id_type=MESH)
                    r.start()
            pltpu.make_async_copy(
                kt_ref.at[pl.ds(HL, HL), pl.ds(0, 128)], k1s_buf,
                cp_sem.at[0]).start()
            pltpu.make_async_copy(
                vt_ref.at[pl.ds(HL, HL), pl.ds(0, 128)], v1s_buf,
                cp_sem.at[1]).start()

        for rb in range(4):
            rows = pl.ds(rb * 512, 512)
            q = jnp.dot(x_ref[0, rows, :].astype(BF16),
                        wq_ref[...].astype(BF16),
                        preferred_element_type=jnp.float32)
            q_buf[rows, :] = q.astype(BF16)

        for d in range(1, N_DEV):
            src = (me + d) % N_DEV
            pltpu.make_async_remote_copy(
                src_ref=qg_buf.at[me], dst_ref=qg_buf.at[me],
                send_sem=qg_send.at[d - 1], recv_sem=qg_recv.at[src],
                device_id=(src,), device_id_type=MESH).wait_recv()

        for r in range(N_DEV):
            ck = pltpu.make_async_copy(
                kt_ref.at[pl.ds(HL * r, HL)], kst, ld_sem.at[0])
            cv = pltpu.make_async_copy(
                vt_ref.at[pl.ds(HL * r, HL)], vst, ld_sem.at[1])
            ck.start()
            cv.start()
            ck.wait()
            cv.wait()
            qr = jnp.transpose(qg_buf[r].reshape(32, HL, DH), (1, 0, 2))
            s = lax.dot_general(qr, kst[...],
                                (((2,), (2,)), ((0,), (0,))),
                                preferred_element_type=jnp.float32) * SCALE
            e = jnp.exp(s)
            lsum = jnp.sum(e, axis=2)
            S = lax.dot_general(e.astype(BF16), vst[...],
                                (((2,), (1,)), ((0,), (0,))),
                                preferred_element_type=jnp.float32)
            sout[r] = S
            lout[r] = lsum

            @pl.when(r == me)
            def _keep_own():
                sin[me] = S
                lin[me] = lsum

            @pl.when(r != me)
            def _send_stats():
                rs = pltpu.make_async_remote_copy(
                    src_ref=sout.at[r], dst_ref=sin.at[me],
                    send_sem=stS_send.at[r], recv_sem=stS_recv.at[me],
                    device_id=(r,), device_id_type=MESH)
                rl = pltpu.make_async_remote_copy(
                    src_ref=lout.at[r], dst_ref=lin.at[me],
                    send_sem=stL_send.at[r],
                    recv_sem=stL_recv.at[me],
                    device_id=(r,), device_id_type=MESH)
                rs.start()
                rl.start()

        for d in range(1, N_DEV):
            src = (me + d) % N_DEV
            pltpu.make_async_remote_copy(
                src_ref=sout.at[0], dst_ref=sin.at[src],
                send_sem=stS_send.at[0], recv_sem=stS_recv.at[src],
                device_id=(src,), device_id_type=MESH).wait_recv()
            pltpu.make_async_remote_copy(
                src_ref=lout.at[0], dst_ref=lin.at[src],
                send_sem=stL_send.at[0], recv_sem=stL_recv.at[src],
                device_id=(src,), device_id_type=MESH).wait_recv()

        @pl.when(me != 1)
        def _wait_sliver():
            pltpu.make_async_remote_copy(
                src_ref=kt_ref.at[pl.ds(0, HL), pl.ds(0, 128)],
                dst_ref=k1s_buf,
                send_sem=k1_send.at[0], recv_sem=k1_recv,
                device_id=(1,), device_id_type=MESH).wait_recv()
            pltpu.make_async_remote_copy(
                src_ref=vt_ref.at[pl.ds(0, HL), pl.ds(0, 128)],
                dst_ref=v1s_buf,
                send_sem=v1_send.at[0], recv_sem=v1_recv,
                device_id=(1,), device_id_type=MESH).wait_recv()

        @pl.when(me == 0)
        def _wait_own_chunk0():
            pltpu.make_async_copy(
                kt_ref.at[pl.ds(0, HL)], k0_buf, cp_sem.at[0]).wait()
            pltpu.make_async_copy(
                vt_ref.at[pl.ds(0, HL)], v0_buf, cp_sem.at[1]).wait()

        @pl.when(me == 1)
        def _wait_own_sliver():
            pltpu.make_async_copy(
                kt_ref.at[pl.ds(HL, HL), pl.ds(0, 128)], k1s_buf,
                cp_sem.at[0]).wait()
            pltpu.make_async_copy(
                vt_ref.at[pl.ds(HL, HL), pl.ds(0, 128)], v1s_buf,
                cp_sem.at[1]).wait()

        for r in waiters:
            r.wait_send()
        for d in range(1, N_DEV):
            dst = (me + d) % N_DEV
            pltpu.make_async_remote_copy(
                src_ref=sout.at[dst], dst_ref=sin.at[me],
                send_sem=stS_send.at[dst], recv_sem=stS_recv.at[me],
                device_id=(dst,), device_id_type=MESH).wait_send()
            pltpu.make_async_remote_copy(
                src_ref=lout.at[dst], dst_ref=lin.at[me],
                send_sem=stL_send.at[dst], recv_sem=stL_recv.at[me],
                device_id=(dst,), device_id_type=MESH).wait_send()

        Ssum = sin[0] + sin[1] + sin[2] + sin[3]
        Lsum = lin[0] + lin[1] + lin[2] + lin[3]
        ctx_g = (Ssum / Lsum[:, :, None]).astype(BF16)
        for hh in range(HL):
            ctx_buf[pl.ds(0, 32), pl.ds(DH * hh, DH)] = ctx_g[hh]

    @pl.when((me != 0) & (h % HG == 0))
    def _wait_k0_group():
        pltpu.make_async_remote_copy(
            src_ref=kt_ref.at[pl.ds(0, HG)],
            dst_ref=k0_buf.at[pl.ds(h, HG)],
            send_sem=k0_send.at[0], recv_sem=k0_recv.at[h // HG],
            device_id=(0,), device_id_type=MESH).wait_recv()
        pltpu.make_async_remote_copy(
            src_ref=vt_ref.at[pl.ds(0, HG)],
            dst_ref=v0_buf.at[pl.ds(h, HG)],
            send_sem=v0_send.at[0], recv_sem=v0_recv.at[h // HG],
            device_id=(0,), device_id_type=MESH).wait_recv()

    k0h = k0_buf[h]
    v0h = v0_buf[h]
    k1h = k1s_buf[h]
    v1h = v1s_buf[h]
    qh = q_buf[:, pl.ds(DH * h, DH)]
    col = pl.ds(DH * h, DH)

    ctx_buf[pl.ds(32, 224), col] = _attend(
        qh[32:256], 32, [(k0h[:512], v0h[:512], 0)]).astype(BF16)
    ctx_buf[pl.ds(256, 256), col] = _attend(
        qh[256:512], 256, [(k0h[:768], v0h[:768], 0)]).astype(BF16)
    for qb in range(2, 7):
        lo = (qb - 1) * BQ
        ctx_buf[pl.ds(qb * BQ, BQ), col] = _attend(
            qh[qb * BQ:(qb + 1) * BQ], qb * BQ,
            [(k0h[:BQ], v0h[:BQ], 0),
             (k0h[lo:lo + 3 * BQ], v0h[lo:lo + 3 * BQ], lo)],
        ).astype(BF16)
    ctx_buf[pl.ds(7 * BQ, BQ), col] = _attend(
        qh[7 * BQ:8 * BQ], 7 * BQ,
        [(k0h[:BQ], v0h[:BQ], 0),
         (k0h[6 * BQ:8 * BQ], v0h[6 * BQ:8 * BQ], 6 * BQ),
         (k1h, v1h, 2048)],
    ).astype(BF16)

    @pl.when((h == HL - 1) & (me == 0))
    def _drain_chunk0_sends():
        for d in range(1, N_DEV):
            for g in range(HL // HG):
                pltpu.make_async_remote_copy(
                    src_ref=kt_ref.at[pl.ds(HL * d + g * HG, HG)],
                    dst_ref=k0_buf.at[pl.ds(g * HG, HG)],
                    send_sem=k0_send.at[(d - 1) * (HL // HG) + g],
                    recv_sem=k0_recv.at[g],
                    device_id=(d,), device_id_type=MESH).wait_send()
                pltpu.make_async_remote_copy(
                    src_ref=vt_ref.at[pl.ds(HL * d + g * HG, HG)],
                    dst_ref=v0_buf.at[pl.ds(g * HG, HG)],
                    send_sem=v0_send.at[(d - 1) * (HL // HG) + g],
                    recv_sem=v0_recv.at[g],
                    device_id=(d,), device_id_type=MESH).wait_send()

    @pl.when((h == HL - 1) & (me == 1))
    def _drain_sliver_sends():
        for d in range(1, N_DEV):
            dst = (1 + d) % N_DEV
            pltpu.make_async_remote_copy(
                src_ref=kt_ref.at[pl.ds(HL * dst, HL), pl.ds(0, 128)],
                dst_ref=k1s_buf,
                send_sem=k1_send.at[d - 1], recv_sem=k1_recv,
                device_id=(dst,), device_id_type=MESH).wait_send()
            pltpu.make_async_remote_copy(
                src_ref=vt_ref.at[pl.ds(HL * dst, HL), pl.ds(0, 128)],
                dst_ref=v1s_buf,
                send_sem=v1_send.at[d - 1], recv_sem=v1_recv,
                device_id=(dst,), device_id_type=MESH).wait_send()

    @pl.when(h == HL - 1)
    def _project_out():
        out = jnp.dot(ctx_buf[...], wo_ref[...].astype(BF16),
                      preferred_element_type=jnp.float32)
        out_ref[...] = out.astype(BF16)


def _attn(x, Wq, Wo, kt, vt):
    return pl.pallas_call(
        _attn_body,
        grid=(HL,),
        in_specs=[
            pl.BlockSpec((1, SQ, 1024), lambda h: (0, 0, 0)),
            pl.BlockSpec((1024, 1024), lambda h: (0, 0)),
            pl.BlockSpec((1024, 1024), lambda h: (0, 0)),
            pl.BlockSpec(memory_space=pl.ANY),
            pl.BlockSpec(memory_space=pl.ANY),
        ],
        out_specs=pl.BlockSpec((SQ, 1024), lambda h: (0, 0)),
        out_shape=jax.ShapeDtypeStruct((SQ, 1024), BF16),
        scratch_shapes=[
            pltpu.VMEM((SQ, HL * DH), BF16),
            pltpu.VMEM((SQ, HL * DH), BF16),
            pltpu.VMEM((HL, SKV_SH, DH), BF16),
            pltpu.VMEM((HL, SKV_SH, DH), BF16),
            pltpu.VMEM((HL, 128, DH), BF16),
            pltpu.VMEM((HL, 128, DH), BF16),
            pltpu.VMEM((N_DEV, 32, HL * DH), BF16),
            pltpu.VMEM((N_DEV, HL, 32, DH), jnp.float32),
            pltpu.VMEM((N_DEV, HL, 32), jnp.float32),
            pltpu.VMEM((N_DEV, HL, 32, DH), jnp.float32),
            pltpu.VMEM((N_DEV, HL, 32), jnp.float32),
            pltpu.VMEM((HL, SKV_SH, DH), BF16),
            pltpu.VMEM((HL, SKV_SH, DH), BF16),
            pltpu.SemaphoreType.DMA((N_DEV - 1,)),
            pltpu.SemaphoreType.DMA((N_DEV,)),
            pltpu.SemaphoreType.DMA(((N_DEV - 1) * (HL // HG),)),
            pltpu.SemaphoreType.DMA((HL // HG,)),
            pltpu.SemaphoreType.DMA(((N_DEV - 1) * (HL // HG),)),
            pltpu.SemaphoreType.DMA((HL // HG,)),
            pltpu.SemaphoreType.DMA((N_DEV - 1,)),
            pltpu.SemaphoreType.DMA,
            pltpu.SemaphoreType.DMA((N_DEV - 1,)),
            pltpu.SemaphoreType.DMA,
            pltpu.SemaphoreType.DMA((N_DEV,)),
            pltpu.SemaphoreType.DMA((N_DEV,)),
            pltpu.SemaphoreType.DMA((N_DEV,)),
            pltpu.SemaphoreType.DMA((N_DEV,)),
            pltpu.SemaphoreType.DMA((2,)),
            pltpu.SemaphoreType.DMA((2,)),
        ],
        compiler_params=pltpu.CompilerParams(
            collective_id=0, vmem_limit_bytes=56 * 1024 * 1024),
    )(x, Wq, Wo, kt, vt)


_CH = SQ // N_DEV


def _ar_body(p_ref, out_ref, rbuf, sbuf, send_s, recv_s):
    me = lax.axis_index("i")
    left = (me - 1) % N_DEV
    right = (me + 1) % N_DEV

    barrier = pltpu.get_barrier_semaphore()
    for nbr in (left, right):
        pl.semaphore_signal(barrier, inc=1, device_id=(nbr,),
                            device_id_type=MESH)
    pl.semaphore_wait(barrier, 2)

    def chunk(ref, c):
        return ref[pl.ds(_CH * c, _CH), :]

    def hop(src_ref, t):
        r = pltpu.make_async_remote_copy(
            src_ref=src_ref,
            dst_ref=rbuf.at[t],
            send_sem=send_s.at[t],
            recv_sem=recv_s.at[t],
            device_id=(right,),
            device_id_type=MESH,
        )
        r.start()
        r.wait()

    hop(p_ref.at[pl.ds(_CH * me, _CH)], 0)
    s = rbuf[0].astype(jnp.float32) + chunk(p_ref, (me - 1) % N_DEV).astype(jnp.float32)
    sbuf[0] = s.astype(BF16)
    hop(sbuf.at[0], 1)
    s = rbuf[1].astype(jnp.float32) + chunk(p_ref, (me - 2) % N_DEV).astype(jnp.float32)
    sbuf[1] = s.astype(BF16)
    hop(sbuf.at[1], 2)
    f = rbuf[2].astype(jnp.float32) + chunk(p_ref, (me + 1) % N_DEV).astype(jnp.float32)
    sbuf[2] = f.astype(BF16)
    out_ref[0, pl.ds(_CH * ((me + 1) % N_DEV), _CH), :] = f

    hop(sbuf.at[2], 3)
    out_ref[0, pl.ds(_CH * me, _CH), :] = rbuf[3].astype(jnp.float32)
    hop(rbuf.at[3], 4)
    out_ref[0, pl.ds(_CH * ((me - 1) % N_DEV), _CH), :] = rbuf[4].astype(jnp.float32)
    hop(rbuf.at[4], 5)
    out_ref[0, pl.ds(_CH * ((me - 2) % N_DEV), _CH), :] = rbuf[5].astype(jnp.float32)


def _allreduce(partial):
    return pl.pallas_call(
        _ar_body,
        in_specs=[pl.BlockSpec(memory_space=pltpu.VMEM)],
        out_specs=pl.BlockSpec(memory_space=pltpu.VMEM),
        out_shape=jax.ShapeDtypeStruct((1, SQ, 1024), jnp.float32),
        scratch_shapes=[
            pltpu.VMEM((6, _CH, 1024), BF16),
            pltpu.VMEM((3, _CH, 1024), BF16),
            pltpu.SemaphoreType.DMA((6,)),
            pltpu.SemaphoreType.DMA((6,)),
        ],
        compiler_params=pltpu.CompilerParams(
            collective_id=1, vmem_limit_bytes=48 * 1024 * 1024),
    )(partial)


def kernel(x, Wq, K_ext, V_ext, Wo):
    kt, vt = _convert(K_ext, V_ext)
    partial = _attn(x, Wq, Wo, kt, vt)
    return _allreduce(partial)


# device time: 346186 ns/iter; 1.0956x vs baseline; 1.0863x over previous
import jax
import jax.numpy as jnp
from jax import lax
from jax.experimental import pallas as pl
from jax.experimental.pallas import tpu as pltpu

N_DEV = 4
HQ = 32
HL = 8
DH = 128
SQ = 2048
SKV_SH = 2048
SKV = 8192
BQ = 256
HG = 2
SCALE = 0.08838834764831843
BF16 = jnp.bfloat16
MESH = pl.DeviceIdType.MESH


_CVT_CHUNK = 512


def _convert_body(k_ref, v_ref, kt_ref, vt_ref):
    kt_ref[...] = jnp.transpose(k_ref[0].astype(BF16), (1, 0, 2))
    vt_ref[...] = jnp.transpose(v_ref[0].astype(BF16), (1, 0, 2))


def _convert(K_ext, V_ext):
    return pl.pallas_call(
        _convert_body,
        grid=(N_DEV, SKV_SH // _CVT_CHUNK),
        in_specs=[
            pl.BlockSpec((1, _CVT_CHUNK, HL, DH), lambda g, c: (0, c, g, 0)),
            pl.BlockSpec((1, _CVT_CHUNK, HL, DH), lambda g, c: (0, c, g, 0)),
        ],
        out_specs=[
            pl.BlockSpec((HL, _CVT_CHUNK, DH), lambda g, c: (g, c, 0)),
            pl.BlockSpec((HL, _CVT_CHUNK, DH), lambda g, c: (g, c, 0)),
        ],
        out_shape=[
            jax.ShapeDtypeStruct((HQ, SKV_SH, DH), BF16),
            jax.ShapeDtypeStruct((HQ, SKV_SH, DH), BF16),
        ],
    )(K_ext, V_ext)


def _attend(qblk, qi0, segs):
    parts = [
        lax.dot_general(qblk, k, (((1,), (1,)), ((), ())),
                        preferred_element_type=jnp.float32)
        for k, _, _ in segs
    ]
    s = jnp.concatenate(parts, axis=1) * SCALE if len(parts) > 1 else parts[0] * SCALE
    ki = jnp.concatenate(
        [ki0 + lax.broadcasted_iota(jnp.int32, (1, k.shape[0]), 1)
         for k, _, ki0 in segs], axis=1)
    qi = qi0 + lax.broadcasted_iota(jnp.int32, s.shape, 0)
    mask = (jnp.abs(qi - ki) <= 128) | (ki < 32) | (qi < 32)
    s = jnp.where(mask, s, -1e9)
    e = jnp.exp(s - jnp.max(s, axis=1, keepdims=True))
    w = (e / jnp.sum(e, axis=1, keepdims=True)).astype(BF16)
    acc = None
    off = 0
    for k, v, _ in segs:
        p = jnp.dot(w[:, off:off + k.shape[0]], v,
                    preferred_element_type=jnp.float32)
        acc = p if acc is None else acc + p
        off += k.shape[0]
    return acc


def _attn_body(x_ref, wq_ref, wo_ref, kt_ref, vt_ref,
               out_ref,
               q_buf, ctx_buf, k0_buf, v0_buf, k1s_buf, v1s_buf,
               qg_buf, sout, lout, sin, lin, kst, vst,
               qg_send, qg_recv, k0_send, k0_recv, v0_send, v0_recv,
               k1_send, k1_recv, v1_send, v1_recv,
               stS_send, stS_recv, stL_send, stL_recv,
               cp_sem, ld_sem):
    h = pl.program_id(0)
    me = lax.axis_index("i")

    @pl.when(h == 0)
    def _comm():
        barrier = pltpu.get_barrier_semaphore()
        for d in range(1, N_DEV):
            pl.semaphore_signal(barrier, inc=1,
                                device_id=((me + d) % N_DEV,),
                                device_id_type=MESH)
        pl.semaphore_wait(barrier, N_DEV - 1)

        waiters = []

        xg = x_ref[0, 0:32, :].astype(BF16)
        qg_buf[me] = jnp.dot(xg, wq_ref[...].astype(BF16),
                             preferred_element_type=jnp.float32).astype(BF16)
        for d in range(1, N_DEV):
            dst = (me + d) % N_DEV
            r = pltpu.make_async_remote_copy(
                src_ref=qg_buf.at[me], dst_ref=qg_buf.at[me],
                send_sem=qg_send.at[d - 1], recv_sem=qg_recv.at[me],
                device_id=(dst,), device_id_type=MESH)
            r.start()
            waiters.append(r)

        @pl.when(me == 0)
        def _send_chunk0():
            for g in range(HL // HG):
                for d in range(1, N_DEV):
                    for src_t, dbuf, ssem, rsem in (
                        (kt_ref, k0_buf, k0_send, k0_recv),
                        (vt_ref, v0_buf, v0_send, v0_recv),
                    ):
                        r = pltpu.make_async_remote_copy(
                            src_ref=src_t.at[pl.ds(HL * d + g * HG, HG)],
                            dst_ref=dbuf.at[pl.ds(g * HG, HG)],
                            send_sem=ssem.at[(d - 1) * (HL // HG) + g],
                            recv_sem=rsem.at[g],
                            device_id=(d,), device_id_type=MESH)
                        r.start()
            pltpu.make_async_copy(
                kt_ref.at[pl.ds(0, HL)], k0_buf, cp_sem.at[0]).start()
            pltpu.make_async_copy(
                vt_ref.at[pl.ds(0, HL)], v0_buf, cp_sem.at[1]).start()

        @pl.when(me == 1)
        def _send_sliver():
            for d in range(1, N_DEV):
                dst = (1 + d) % N_DEV
                for src_t, dbuf, ssem, rsem in (
                    (kt_ref, k1s_buf, k1_send, k1_recv),
                    (vt_ref, v1s_buf, v1_send, v1_recv),
                ):
                    r = pltpu.make_async_remote_copy(
                        src_ref=src_t.at[pl.ds(HL * dst, HL), pl.ds(0, 128)],
                        dst_ref=dbuf,
                        send_sem=ssem.at[d - 1], recv_sem=rsem,
                        device_id=(dst,), device_id_type=MESH)
                    r.start()
            pltpu.make_async_copy(
                kt_ref.at[pl.ds(HL, HL), pl.ds(0, 128)], k1s_buf,
                cp_sem.at[0]).start()
            pltpu.make_async_copy(
                vt_ref.at[pl.ds(HL, HL), pl.ds(0, 128)], v1s_buf,
                cp_sem.at[1]).start()

        for rb in range(4):
            rows = pl.ds(rb * 512, 512)
            q = jnp.dot(x_ref[0, rows, :].astype(BF16),
                        wq_ref[...].astype(BF16),
                        preferred_element_type=jnp.float32)
            q_buf[rows, :] = q.astype(BF16)

        for d in range(1, N_DEV):
            src = (me + d) % N_DEV
            pltpu.make_async_remote_copy(
                src_ref=qg_buf.at[me], dst_ref=qg_buf.at[me],
                send_sem=qg_send.at[d - 1], recv_sem=qg_recv.at[src],
                device_id=(src,), device_id_type=MESH).wait_recv()

        for r in range(N_DEV):
            ck = pltpu.make_async_copy(
                kt_ref.at[pl.ds(HL * r, HL)], kst, ld_sem.at[0])
            cv = pltpu.make_async_copy(
                vt_ref.at[pl.ds(HL * r, HL)], vst, ld_sem.at[1])
            ck.start()
            cv.start()
            ck.wait()
            cv.wait()
            qr = jnp.transpose(qg_buf[r].reshape(32, HL, DH), (1, 0, 2))
            s = lax.dot_general(qr, kst[...],
                                (((2,), (2,)), ((0,), (0,))),
                                preferred_element_type=jnp.float32) * SCALE
            e = jnp.exp(s)
            lsum = jnp.sum(e, axis=2)
            S = lax.dot_general(e.astype(BF16), vst[...],
                                (((2,), (1,)), ((0,), (0,))),
                                preferred_element_type=jnp.float32)
            sout[r] = S
            lout[r] = lsum

            @pl.when(r == me)
            def _keep_own():
                sin[me] = S
                lin[me] = lsum

            @pl.when(r != me)
            def _send_stats():
                rs = pltpu.make_async_remote_copy(
                    src_ref=sout.at[r], dst_ref=sin.at[me],
                    send_sem=stS_send.at[r], recv_sem=stS_recv.at[me],
                    device_id=(r,), device_id_type=MESH)
                rl = pltpu.make_async_remote_copy(
                    src_ref=lout.at[r], dst_ref=lin.at[me],
                    send_sem=stL_send.at[r],
                    recv_sem=stL_recv.at[me],
                    device_id=(r,), device_id_type=MESH)
                rs.start()
                rl.start()

        for d in range(1, N_DEV):
            src = (me + d) % N_DEV
            pltpu.make_async_remote_copy(
                src_ref=sout.at[0], dst_ref=sin.at[src],
                send_sem=stS_send.at[0], recv_sem=stS_recv.at[src],
                device_id=(src,), device_id_type=MESH).wait_recv()
            pltpu.make_async_remote_copy(
                src_ref=lout.at[0], dst_ref=lin.at[src],
                send_sem=stL_send.at[0], recv_sem=stL_recv.at[src],
                device_id=(src,), device_id_type=MESH).wait_recv()

        @pl.when(me != 1)
        def _wait_sliver():
            pltpu.make_async_remote_copy(
                src_ref=kt_ref.at[pl.ds(0, HL), pl.ds(0, 128)],
                dst_ref=k1s_buf,
                send_sem=k1_send.at[0], recv_sem=k1_recv,
                device_id=(1,), device_id_type=MESH).wait_recv()
            pltpu.make_async_remote_copy(
                src_ref=vt_ref.at[pl.ds(0, HL), pl.ds(0, 128)],
                dst_ref=v1s_buf,
                send_sem=v1_send.at[0], recv_sem=v1_recv,
                device_id=(1,), device_id_type=MESH).wait_recv()

        @pl.when(me == 0)
        def _wait_own_chunk0():
            pltpu.make_async_copy(
                kt_ref.at[pl.ds(0, HL)], k0_buf, cp_sem.at[0]).wait()
            pltpu.make_async_copy(
                vt_ref.at[pl.ds(0, HL)], v0_buf, cp_sem.at[1]).wait()

        @pl.when(me == 1)
        def _wait_own_sliver():
            pltpu.make_async_copy(
                kt_ref.at[pl.ds(HL, HL), pl.ds(0, 128)], k1s_buf,
                cp_sem.at[0]).wait()
            pltpu.make_async_copy(
                vt_ref.at[pl.ds(HL, HL), pl.ds(0, 128)], v1s_buf,
                cp_sem.at[1]).wait()

        for r in waiters:
            r.wait_send()
        for d in range(1, N_DEV):
            dst = (me + d) % N_DEV
            pltpu.make_async_remote_copy(
                src_ref=sout.at[dst], dst_ref=sin.at[me],
                send_sem=stS_send.at[dst], recv_sem=stS_recv.at[me],
                device_id=(dst,), device_id_type=MESH).wait_send()
            pltpu.make_async_remote_copy(
                src_ref=lout.at[dst], dst_ref=lin.at[me],
                send_sem=stL_send.at[dst], recv_sem=stL_recv.at[me],
                device_id=(dst,), device_id_type=MESH).wait_send()

        Ssum = sin[0] + sin[1] + sin[2] + sin[3]
        Lsum = lin[0] + lin[1] + lin[2] + lin[3]
        ctx_g = (Ssum / Lsum[:, :, None]).astype(BF16)
        for hh in range(HL):
            ctx_buf[pl.ds(0, 32), pl.ds(DH * hh, DH)] = ctx_g[hh]

    @pl.when((me != 0) & (h % HG == 0))
    def _wait_k0_group():
        pltpu.make_async_remote_copy(
            src_ref=kt_ref.at[pl.ds(0, HG)],
            dst_ref=k0_buf.at[pl.ds(h, HG)],
            send_sem=k0_send.at[0], recv_sem=k0_recv.at[h // HG],
            device_id=(0,), device_id_type=MESH).wait_recv()
        pltpu.make_async_remote_copy(
            src_ref=vt_ref.at[pl.ds(0, HG)],
            dst_ref=v0_buf.at[pl.ds(h, HG)],
            send_sem=v0_send.at[0], recv_sem=v0_recv.at[h // HG],
            device_id=(0,), device_id_type=MESH).wait_recv()

    k0h = k0_buf[h]
    v0h = v0_buf[h]
    k1h = k1s_buf[h]
    v1h = v1s_buf[h]
    qh = q_buf[:, pl.ds(DH * h, DH)]
    col = pl.ds(DH * h, DH)

    ctx_buf[pl.ds(32, 224), col] = _attend(
        qh[32:256], 32, [(k0h[:512], v0h[:512], 0)]).astype(BF16)
    ctx_buf[pl.ds(256, 256), col] = _attend(
        qh[256:512], 256, [(k0h[:768], v0h[:768], 0)]).astype(BF16)
    for qb in range(2, 7):
        lo = (qb - 1) * BQ
        ctx_buf[pl.ds(qb * BQ, BQ), col] = _attend(
            qh[qb * BQ:(qb + 1) * BQ], qb * BQ,
            [(k0h[:BQ], v0h[:BQ], 0),
             (k0h[lo:lo + 3 * BQ], v0h[lo:lo + 3 * BQ], lo)],
        ).astype(BF16)
    ctx_buf[pl.ds(7 * BQ, BQ), col] = _attend(
        qh[7 * BQ:8 * BQ], 7 * BQ,
        [(k0h[:BQ], v0h[:BQ], 0),
         (k0h[6 * BQ:8 * BQ], v0h[6 * BQ:8 * BQ], 6 * BQ),
         (k1h, v1h, 2048)],
    ).astype(BF16)

    @pl.when((h == HL - 1) & (me == 0))
    def _drain_chunk0_sends():
        for d in range(1, N_DEV):
            for g in range(HL // HG):
                pltpu.make_async_remote_copy(
                    src_ref=kt_ref.at[pl.ds(HL * d + g * HG, HG)],
                    dst_ref=k0_buf.at[pl.ds(g * HG, HG)],
                    send_sem=k0_send.at[(d - 1) * (HL // HG) + g],
                    recv_sem=k0_recv.at[g],
                    device_id=(d,), device_id_type=MESH).wait_send()
                pltpu.make_async_remote_copy(
                    src_ref=vt_ref.at[pl.ds(HL * d + g * HG, HG)],
                    dst_ref=v0_buf.at[pl.ds(g * HG, HG)],
                    send_sem=v0_send.at[(d - 1) * (HL // HG) + g],
                    recv_sem=v0_recv.at[g],
                    device_id=(d,), device_id_type=MESH).wait_send()

    @pl.when((h == HL - 1) & (me == 1))
    def _drain_sliver_sends():
        for d in range(1, N_DEV):
            dst = (1 + d) % N_DEV
            pltpu.make_async_remote_copy(
                src_ref=kt_ref.at[pl.ds(HL * dst, HL), pl.ds(0, 128)],
                dst_ref=k1s_buf,
                send_sem=k1_send.at[d - 1], recv_sem=k1_recv,
                device_id=(dst,), device_id_type=MESH).wait_send()
            pltpu.make_async_remote_copy(
                src_ref=vt_ref.at[pl.ds(HL * dst, HL), pl.ds(0, 128)],
                dst_ref=v1s_buf,
                send_sem=v1_send.at[d - 1], recv_sem=v1_recv,
                device_id=(dst,), device_id_type=MESH).wait_send()

    @pl.when(h == HL - 1)
    def _project_out():
        out = jnp.dot(ctx_buf[...], wo_ref[...].astype(BF16),
                      preferred_element_type=jnp.float32)
        out_ref[...] = out.astype(BF16)


def _attn(x, Wq, Wo, kt, vt):
    return pl.pallas_call(
        _attn_body,
        grid=(HL,),
        in_specs=[
            pl.BlockSpec((1, SQ, 1024), lambda h: (0, 0, 0)),
            pl.BlockSpec((1024, 1024), lambda h: (0, 0)),
            pl.BlockSpec((1024, 1024), lambda h: (0, 0)),
            pl.BlockSpec(memory_space=pl.ANY),
            pl.BlockSpec(memory_space=pl.ANY),
        ],
        out_specs=pl.BlockSpec((SQ, 1024), lambda h: (0, 0)),
        out_shape=jax.ShapeDtypeStruct((SQ, 1024), BF16),
        scratch_shapes=[
            pltpu.VMEM((SQ, HL * DH), BF16),
            pltpu.VMEM((SQ, HL * DH), BF16),
            pltpu.VMEM((HL, SKV_SH, DH), BF16),
            pltpu.VMEM((HL, SKV_SH, DH), BF16),
            pltpu.VMEM((HL, 128, DH), BF16),
            pltpu.VMEM((HL, 128, DH), BF16),
            pltpu.VMEM((N_DEV, 32, HL * DH), BF16),
            pltpu.VMEM((N_DEV, HL, 32, DH), jnp.float32),
            pltpu.VMEM((N_DEV, HL, 32), jnp.float32),
            pltpu.VMEM((N_DEV, HL, 32, DH), jnp.float32),
            pltpu.VMEM((N_DEV, HL, 32), jnp.float32),
            pltpu.VMEM((HL, SKV_SH, DH), BF16),
            pltpu.VMEM((HL, SKV_SH, DH), BF16),
            pltpu.SemaphoreType.DMA((N_DEV - 1,)),
            pltpu.SemaphoreType.DMA((N_DEV,)),
            pltpu.SemaphoreType.DMA(((N_DEV - 1) * (HL // HG),)),
            pltpu.SemaphoreType.DMA((HL // HG,)),
            pltpu.SemaphoreType.DMA(((N_DEV - 1) * (HL // HG),)),
            pltpu.SemaphoreType.DMA((HL // HG,)),
            pltpu.SemaphoreType.DMA((N_DEV - 1,)),
            pltpu.SemaphoreType.DMA,
            pltpu.SemaphoreType.DMA((N_DEV - 1,)),
            pltpu.SemaphoreType.DMA,
            pltpu.SemaphoreType.DMA((N_DEV,)),
            pltpu.SemaphoreType.DMA((N_DEV,)),
            pltpu.SemaphoreType.DMA((N_DEV,)),
            pltpu.SemaphoreType.DMA((N_DEV,)),
            pltpu.SemaphoreType.DMA((2,)),
            pltpu.SemaphoreType.DMA((2,)),
        ],
        compiler_params=pltpu.CompilerParams(
            collective_id=0, vmem_limit_bytes=56 * 1024 * 1024),
    )(x, Wq, Wo, kt, vt)


_CH = SQ // N_DEV


_HC = 512


def _ar_body(p_ref, out_ref, rbufR, rbufL, sbufR, sbufL,
             sR_send, sR_recv, sL_send, sL_recv):
    me = lax.axis_index("i")
    left = (me - 1) % N_DEV
    right = (me + 1) % N_DEV

    barrier = pltpu.get_barrier_semaphore()
    for nbr in (left, right):
        pl.semaphore_signal(barrier, inc=1, device_id=(nbr,),
                            device_id_type=MESH)
    pl.semaphore_wait(barrier, 2)

    CR = pl.ds(0, _HC)
    CL = pl.ds(_HC, _HC)

    def rows(c):
        return pl.ds(_CH * ((me + c) % N_DEV), _CH)

    def hops(srcR, srcL, t):
        rR = pltpu.make_async_remote_copy(
            src_ref=srcR, dst_ref=rbufR.at[t],
            send_sem=sR_send.at[t], recv_sem=sR_recv.at[t],
            device_id=(right,), device_id_type=MESH)
        rL = pltpu.make_async_remote_copy(
            src_ref=srcL, dst_ref=rbufL.at[t],
            send_sem=sL_send.at[t], recv_sem=sL_recv.at[t],
            device_id=(left,), device_id_type=MESH)
        rR.start()
        rL.start()
        rR.wait()
        rL.wait()

    hops(p_ref.at[rows(0), CR], p_ref.at[rows(0), CL], 0)
    sbufR[0] = (rbufR[0].astype(jnp.float32)
                + p_ref[rows(-1), CR].astype(jnp.float32)).astype(BF16)
    sbufL[0] = (rbufL[0].astype(jnp.float32)
                + p_ref[rows(1), CL].astype(jnp.float32)).astype(BF16)
    hops(sbufR.at[0], sbufL.at[0], 1)
    sbufR[1] = (rbufR[1].astype(jnp.float32)
                + p_ref[rows(-2), CR].astype(jnp.float32)).astype(BF16)
    sbufL[1] = (rbufL[1].astype(jnp.float32)
                + p_ref[rows(2), CL].astype(jnp.float32)).astype(BF16)
    hops(sbufR.at[1], sbufL.at[1], 2)
    fR = rbufR[2].astype(jnp.float32) + p_ref[rows(1), CR].astype(jnp.float32)
    fL = rbufL[2].astype(jnp.float32) + p_ref[rows(-1), CL].astype(jnp.float32)
    sbufR[2] = fR.astype(BF16)
    sbufL[2] = fL.astype(BF16)
    out_ref[0, rows(1), CR] = fR
    out_ref[0, rows(-1), CL] = fL

    hops(sbufR.at[2], sbufL.at[2], 3)
    out_ref[0, rows(0), CR] = rbufR[3].astype(jnp.float32)
    out_ref[0, rows(0), CL] = rbufL[3].astype(jnp.float32)
    hops(rbufR.at[3], rbufL.at[3], 4)
    out_ref[0, rows(-1), CR] = rbufR[4].astype(jnp.float32)
    out_ref[0, rows(1), CL] = rbufL[4].astype(jnp.float32)
    hops(rbufR.at[4], rbufL.at[4], 5)
    out_ref[0, rows(-2), CR] = rbufR[5].astype(jnp.float32)
    out_ref[0, rows(2), CL] = rbufL[5].astype(jnp.float32)


def _allreduce(partial):
    return pl.pallas_call(
        _ar_body,
        in_specs=[pl.BlockSpec(memory_space=pltpu.VMEM)],
        out_specs=pl.BlockSpec(memory_space=pltpu.VMEM),
        out_shape=jax.ShapeDtypeStruct((1, SQ, 1024), jnp.float32),
        scratch_shapes=[
            pltpu.VMEM((6, _CH, _HC), BF16),
            pltpu.VMEM((6, _CH, _HC), BF16),
            pltpu.VMEM((3, _CH, _HC), BF16),
            pltpu.VMEM((3, _CH, _HC), BF16),
            pltpu.SemaphoreType.DMA((6,)),
            pltpu.SemaphoreType.DMA((6,)),
            pltpu.SemaphoreType.DMA((6,)),
            pltpu.SemaphoreType.DMA((6,)),
        ],
        compiler_params=pltpu.CompilerParams(
            collective_id=1, vmem_limit_bytes=48 * 1024 * 1024),
    )(partial)


def kernel(x, Wq, K_ext, V_ext, Wo):
    kt, vt = _convert(K_ext, V_ext)
    partial = _attn(x, Wq, Wo, kt, vt)
    return _allreduce(partial)
